# baseline (speedup 1.0000x reference)
import functools

import jax
import jax.numpy as jnp
from jax import lax
from jax.experimental import pallas as pl
from jax.experimental.pallas import tpu as pltpu
from jax.experimental.pallas import tpu_sc as plsc

_N = 10000
_NP = 10240
_EP = 163840
_SUB = 128
_NSUB = 40
_TILE_E = _SUB * _NSUB
_PAD_IDX = 10200


def _mm_body(a_ref, b_ref, o_ref):
    o_ref[...] = jnp.dot(a_ref[...], b_ref[...],
                         preferred_element_type=jnp.float32)


def _mm(a, b, bm=400):
    n, kdim = a.shape
    m = b.shape[1]
    return pl.pallas_call(
        _mm_body,
        grid=(n // bm,),
        in_specs=[pl.BlockSpec((bm, kdim), lambda i: (i, 0)),
                  pl.BlockSpec((kdim, m), lambda i: (0, 0))],
        out_specs=pl.BlockSpec((bm, m), lambda i: (i, 0)),
        out_shape=jax.ShapeDtypeStruct((n, m), jnp.float32),
    )(a, b)


def _mm_rowsum_body(a_ref, b_ref, o_ref, rs_ref):
    a = a_ref[...]
    o_ref[...] = jnp.dot(a, b_ref[...], preferred_element_type=jnp.float32)
    rs_ref[...] = jnp.broadcast_to(jnp.sum(a, axis=1, keepdims=True),
                                   rs_ref.shape)


def _mm_rowsum(a, b, bm=400):
    n, kdim = a.shape
    m = b.shape[1]
    return pl.pallas_call(
        _mm_rowsum_body,
        grid=(n // bm,),
        in_specs=[pl.BlockSpec((bm, kdim), lambda i: (i, 0)),
                  pl.BlockSpec((kdim, m), lambda i: (0, 0))],
        out_specs=[pl.BlockSpec((bm, m), lambda i: (i, 0)),
                   pl.BlockSpec((bm, 8), lambda i: (i, 0))],
        out_shape=[jax.ShapeDtypeStruct((n, m), jnp.float32),
                   jax.ShapeDtypeStruct((n, 8), jnp.float32)],
    )(a, b)


def _proj_body(x_ref, w_ref, o_ref, nm_ref):
    i = pl.program_id(0)
    y = jnp.dot(x_ref[...], w_ref[...], preferred_element_type=jnp.float32)
    o_ref[...] = y
    blk = jnp.full(nm_ref.shape, jnp.max(jnp.sum(y * y, axis=1)))

    @pl.when(i == 0)
    def _():
        nm_ref[...] = blk

    @pl.when(i != 0)
    def _():
        nm_ref[...] = jnp.maximum(nm_ref[...], blk)


def _proj(x, wt, bm=512):
    n, kdim = x.shape
    m = wt.shape[1]
    return pl.pallas_call(
        _proj_body,
        grid=(n // bm,),
        in_specs=[pl.BlockSpec((bm, kdim), lambda i: (i, 0)),
                  pl.BlockSpec((kdim, m), lambda i: (0, 0))],
        out_specs=[pl.BlockSpec((bm, m), lambda i: (i, 0)),
                   pl.BlockSpec((8, 128), lambda i: (0, 0))],
        out_shape=[jax.ShapeDtypeStruct((n, m), jnp.float32),
                   jax.ShapeDtypeStruct((8, 128), jnp.float32)],
    )(x, wt)


def _recip_body(s_ref, o_ref):
    o_ref[...] = 1.0 / (s_ref[0:1, :] + s_ref[1:2, :] + 1e-16)


def _recip(s_part):
    out = pl.pallas_call(
        _recip_body,
        out_shape=jax.ShapeDtypeStruct((1, _NP), jnp.float32),
    )(s_part)
    return out.reshape(_NP)


def _add2_body(a_ref, b_ref, o_ref):
    o_ref[...] = a_ref[0] + b_ref[0]


def _add2(p, bm=2048):
    _, n, m = p.shape
    return pl.pallas_call(
        _add2_body,
        grid=(n // bm,),
        in_specs=[pl.BlockSpec((1, bm, m), lambda i: (0, i, 0)),
                  pl.BlockSpec((1, bm, m), lambda i: (1, i, 0))],
        out_specs=pl.BlockSpec((bm, m), lambda i: (i, 0)),
        out_shape=jax.ShapeDtypeStruct((n, m), jnp.float32),
    )(p, p)


@functools.lru_cache(maxsize=None)
def _gat_phase1(dd):
    mesh = plsc.VectorSubcoreMesh(core_axis_name="c", subcore_axis_name="s")

    def body(xl_hbm, xr_hbm, src_hbm, dst_hbm, attb_hbm, mg_hbm, zs_hbm,
             ex_hbm, s_part_hbm,
             attb_v, mg_v, src_v, dst_v, xl_rows, xr_rows, ex_v, s_sh,
             sem1, sem2):
        c = lax.axis_index("c")
        s = lax.axis_index("s")
        wid = s * 2 + c
        pltpu.sync_copy(attb_hbm, attb_v)
        pltpu.sync_copy(mg_hbm, mg_v)
        pltpu.sync_copy(zs_hbm, s_sh.at[pl.ds(s * 640, 640)])
        plsc.subcore_barrier()
        mgv = mg_v[...]
        iota = lax.iota(jnp.int32, 16)

        def subchunk(j, carry):
            base = wid * _TILE_E + j * _SUB
            pltpu.sync_copy(src_hbm.at[pl.ds(base, _SUB)], src_v)
            pltpu.sync_copy(dst_hbm.at[pl.ds(base, _SUB)], dst_v)
            pltpu.async_copy(xl_hbm.at[src_v], xl_rows, sem1).wait()
            pltpu.async_copy(xr_hbm.at[dst_v], xr_rows, sem2).wait()

            def group(g, carry2):
                rows = iota + g * 16

                def dstep(d, acc):
                    dvec = jnp.full((16,), d, jnp.int32)
                    a = plsc.load_gather(xl_rows, [rows, dvec])
                    b = plsc.load_gather(xr_rows, [rows, dvec])
                    v = a + b
                    lr = jnp.maximum(v, 0.2 * v)
                    return acc + lr * attb_v[pl.ds(d * 16, 16)]

                logit = lax.fori_loop(0, dd, dstep,
                                      jnp.zeros((16,), jnp.float32),
                                      unroll=16)
                ex_v[pl.ds(g * 16, 16)] = jnp.exp(logit - mgv)
                return carry2

            lax.fori_loop(0, _SUB // 16, group, 0)
            pltpu.sync_copy(ex_v, ex_hbm.at[pl.ds(base, _SUB)])
            pltpu.sync_copy(ex_v, s_sh.at[dst_v], add=True)
            return carry

        lax.fori_loop(0, _NSUB, subchunk, 0)
        plsc.subcore_barrier()

        @pl.when(s == 0)
        def _():
            pltpu.sync_copy(s_sh, s_part_hbm.at[c])

    return pl.kernel(
        body,
        compiler_params=pltpu.CompilerParams(use_tc_tiling_on_sc=False,
                                             needs_layout_passes=False),
        out_type=[jax.ShapeDtypeStruct((_EP,), jnp.float32),
                  jax.ShapeDtypeStruct((2, _NP), jnp.float32)],
        mesh=mesh,
        scratch_types=[
            pltpu.VMEM((dd * 16,), jnp.float32),
            pltpu.VMEM((16,), jnp.float32),
            pltpu.VMEM((_SUB,), jnp.int32),
            pltpu.VMEM((_SUB,), jnp.int32),
            pltpu.VMEM((_SUB, dd), jnp.float32),
            pltpu.VMEM((_SUB, dd), jnp.float32),
            pltpu.VMEM((_SUB,), jnp.float32),
            pltpu.VMEM_SHARED((_NP,), jnp.float32),
            pltpu.SemaphoreType.DMA,
            pltpu.SemaphoreType.DMA,
        ],
    )


@functools.lru_cache(maxsize=None)
def _gat_phase2(dd):
    mesh = plsc.VectorSubcoreMesh(core_axis_name="c", subcore_axis_name="s")

    def body(xl_hbm, src_hbm, dst_hbm, ex_hbm, r_hbm, zo_hbm,
             out_part_hbm,
             r_loc, src_v, dst_v, ex_v, xl_rows, scaled, out_sh, sem1):
        c = lax.axis_index("c")
        s = lax.axis_index("s")
        wid = s * 2 + c
        pltpu.sync_copy(r_hbm, r_loc)
        pltpu.sync_copy(zo_hbm, out_sh.at[pl.ds(s * 640, 640), :])
        plsc.subcore_barrier()
        iota = lax.iota(jnp.int32, 16)

        def subchunk(j, carry):
            base = wid * _TILE_E + j * _SUB
            pltpu.sync_copy(src_hbm.at[pl.ds(base, _SUB)], src_v)
            pltpu.sync_copy(dst_hbm.at[pl.ds(base, _SUB)], dst_v)
            pltpu.sync_copy(ex_hbm.at[pl.ds(base, _SUB)], ex_v)
            pltpu.async_copy(xl_hbm.at[src_v], xl_rows, sem1).wait()

            def group(g, carry2):
                rows = iota + g * 16
                dstg = dst_v[pl.ds(g * 16, 16)]
                alpha = ex_v[pl.ds(g * 16, 16)] * plsc.load_gather(r_loc,
                                                                   [dstg])

                def dstep(d, carry3):
                    dvec = jnp.full((16,), d, jnp.int32)
                    valv = plsc.load_gather(xl_rows, [rows, dvec])
                    plsc.store_scatter(scaled, [rows, dvec], valv * alpha)
                    return carry3

                lax.fori_loop(0, dd, dstep, 0, unroll=16)
                return carry2

            lax.fori_loop(0, _SUB // 16, group, 0)
            pltpu.sync_copy(scaled, out_sh.at[dst_v], add=True)
            return carry

        lax.fori_loop(0, _NSUB, subchunk, 0)
        plsc.subcore_barrier()
        pltpu.sync_copy(out_sh.at[pl.ds(s * 640, 640), :],
                        out_part_hbm.at[c, pl.ds(s * 640, 640), :])

    return pl.kernel(
        body,
        compiler_params=pltpu.CompilerParams(use_tc_tiling_on_sc=False,
                                             needs_layout_passes=False),
        out_type=jax.ShapeDtypeStruct((2, _NP, dd), jnp.float32),
        mesh=mesh,
        scratch_types=[
            pltpu.VMEM((_NP,), jnp.float32),
            pltpu.VMEM((_SUB,), jnp.int32),
            pltpu.VMEM((_SUB,), jnp.int32),
            pltpu.VMEM((_SUB,), jnp.float32),
            pltpu.VMEM((_SUB, dd), jnp.float32),
            pltpu.VMEM((_SUB, dd), jnp.float32),
            pltpu.VMEM_SHARED((_NP, dd), jnp.float32),
            pltpu.SemaphoreType.DMA,
        ],
    )


def _gatv2_sc(xl, xr, mg, att, srcp, dstp):
    dd = xl.shape[1]
    attb = jnp.repeat(att, 16)
    mg16 = jnp.full((16,), mg, jnp.float32)
    zs = jnp.zeros((640,), jnp.float32)
    zo = jnp.zeros((640, dd), jnp.float32)
    ex, s_part = _gat_phase1(dd)(xl, xr, srcp, dstp, attb, mg16, zs)
    r = _recip(s_part)
    out_part = _gat_phase2(dd)(xl, srcp, dstp, ex, r, zo)
    return _add2(out_part)


def kernel(feat, feat_a, graph_neigh, edge_index, adj,
           Wl_zip, Wr_zip, att_zip, Wl_eco, Wr_eco, att_eco, W_disc, b_disc):
    n = feat.shape[0]
    e = edge_index.shape[1]
    srcp = jnp.concatenate(
        [edge_index[0], jnp.full((_EP - e,), _PAD_IDX, jnp.int32)])
    dstp = jnp.concatenate(
        [edge_index[1], jnp.full((_EP - e,), _PAD_IDX, jnp.int32)])

    featp = jnp.pad(feat, ((0, _NP - n), (0, 0)))
    featap = jnp.pad(feat_a, ((0, _NP - n), (0, 0)))

    att_nz = jnp.sqrt(jnp.sum(att_zip * att_zip))
    xl1, nl1 = _proj(featp, Wl_zip.T)
    xr1, nr1 = _proj(featp, Wr_zip.T)
    xl2, nl2 = _proj(featap, Wl_zip.T)
    xr2, nr2 = _proj(featap, Wr_zip.T)
    mg1 = att_nz * (jnp.sqrt(jnp.max(nl1)) + jnp.sqrt(jnp.max(nr1)))
    mg2 = att_nz * (jnp.sqrt(jnp.max(nl2)) + jnp.sqrt(jnp.max(nr2)))

    z = _gatv2_sc(xl1, xr1, mg1, att_zip, srcp, dstp)
    z_a = _gatv2_sc(xl2, xr2, mg2, att_zip, srcp, dstp)

    zz = _mm(adj, jnp.concatenate([z[:n], z_a[:n]], axis=1))
    z2 = zz[:, :64]
    z_a2 = zz[:, 64:]

    z2p = jnp.pad(z2, ((0, _NP - n), (0, 0)))
    att_ne = jnp.sqrt(jnp.sum(att_eco * att_eco))
    xl3, nl3 = _proj(z2p, Wl_eco.T)
    xr3, nr3 = _proj(z2p, Wr_eco.T)
    mg3 = att_ne * (jnp.sqrt(jnp.max(nl3)) + jnp.sqrt(jnp.max(nr3)))
    h = _gatv2_sc(xl3, xr3, mg3, att_eco, srcp, dstp)
    h2 = _mm(adj, h[:n])

    emb = jax.nn.relu(z2)
    emb_a = jax.nn.relu(z_a2)

    vs, rs = _mm_rowsum(graph_neigh, jnp.concatenate([emb, emb_a], axis=1))
    row_sum = rs[:, :1]

    def _finish_readout(ge):
        ge = ge / row_sum
        norm = jnp.sqrt(jnp.sum(ge * ge, axis=1, keepdims=True))
        return jax.nn.sigmoid(ge / jnp.maximum(norm, 1e-12))

    g = _finish_readout(vs[:, :64])
    g_a = _finish_readout(vs[:, 64:])

    a1 = emb @ W_disc
    a2 = emb_a @ W_disc
    sc_1 = jnp.sum(a1 * g, axis=1, keepdims=True) + b_disc
    sc_2 = jnp.sum(a2 * g, axis=1, keepdims=True) + b_disc
    sc_1a = jnp.sum(a2 * g_a, axis=1, keepdims=True) + b_disc
    sc_2a = jnp.sum(a1 * g_a, axis=1, keepdims=True) + b_disc
    ret = jax.nn.sigmoid(jnp.concatenate([sc_1, sc_2], axis=1))
    ret_a = jax.nn.sigmoid(jnp.concatenate([sc_1a, sc_2a], axis=1))

    return (z2, h2, ret, ret_a)


# speedup vs baseline: 1.3833x; 1.3833x over previous
import functools

import jax
import jax.numpy as jnp
from jax import lax
from jax.experimental import pallas as pl
from jax.experimental.pallas import tpu as pltpu
from jax.experimental.pallas import tpu_sc as plsc

_N = 10000
_NP = 10240
_EP = 163840
_SUB = 128
_NSUB = 40
_TILE_E = _SUB * _NSUB
_PAD_IDX = 10200


def _mm_body(a_ref, b_ref, o_ref):
    o_ref[...] = jnp.dot(a_ref[...], b_ref[...],
                         preferred_element_type=jnp.float32)


def _mm(a, b, bm=400):
    n, kdim = a.shape
    m = b.shape[1]
    return pl.pallas_call(
        _mm_body,
        grid=(n // bm,),
        in_specs=[pl.BlockSpec((bm, kdim), lambda i: (i, 0)),
                  pl.BlockSpec((kdim, m), lambda i: (0, 0))],
        out_specs=pl.BlockSpec((bm, m), lambda i: (i, 0)),
        out_shape=jax.ShapeDtypeStruct((n, m), jnp.float32),
    )(a, b)


def _mm_rowsum_body(a_ref, b_ref, o_ref, rs_ref):
    a = a_ref[...]
    o_ref[...] = jnp.dot(a, b_ref[...], preferred_element_type=jnp.float32)
    rs_ref[...] = jnp.broadcast_to(jnp.sum(a, axis=1, keepdims=True),
                                   rs_ref.shape)


def _mm_rowsum(a, b, bm=400):
    n, kdim = a.shape
    m = b.shape[1]
    return pl.pallas_call(
        _mm_rowsum_body,
        grid=(n // bm,),
        in_specs=[pl.BlockSpec((bm, kdim), lambda i: (i, 0)),
                  pl.BlockSpec((kdim, m), lambda i: (0, 0))],
        out_specs=[pl.BlockSpec((bm, m), lambda i: (i, 0)),
                   pl.BlockSpec((bm, 8), lambda i: (i, 0))],
        out_shape=[jax.ShapeDtypeStruct((n, m), jnp.float32),
                   jax.ShapeDtypeStruct((n, 8), jnp.float32)],
    )(a, b)


def _proj_body(x_ref, w_ref, o_ref, nm_ref):
    i = pl.program_id(0)
    y = jnp.dot(x_ref[...], w_ref[...], preferred_element_type=jnp.float32)
    o_ref[...] = y
    blk = jnp.full(nm_ref.shape, jnp.max(jnp.sum(y * y, axis=1)))

    @pl.when(i == 0)
    def _():
        nm_ref[...] = blk

    @pl.when(i != 0)
    def _():
        nm_ref[...] = jnp.maximum(nm_ref[...], blk)


def _proj(x, wt, bm=512):
    n, kdim = x.shape
    m = wt.shape[1]
    return pl.pallas_call(
        _proj_body,
        grid=(n // bm,),
        in_specs=[pl.BlockSpec((bm, kdim), lambda i: (i, 0)),
                  pl.BlockSpec((kdim, m), lambda i: (0, 0))],
        out_specs=[pl.BlockSpec((bm, m), lambda i: (i, 0)),
                   pl.BlockSpec((8, 128), lambda i: (0, 0))],
        out_shape=[jax.ShapeDtypeStruct((n, m), jnp.float32),
                   jax.ShapeDtypeStruct((8, 128), jnp.float32)],
    )(x, wt)


def _recip_body(s_ref, o_ref):
    o_ref[...] = 1.0 / (s_ref[0:1, :] + s_ref[1:2, :] + 1e-16)


def _recip(s_part):
    out = pl.pallas_call(
        _recip_body,
        out_shape=jax.ShapeDtypeStruct((1, _NP), jnp.float32),
    )(s_part)
    return out.reshape(_NP)


def _add2r_body(a_ref, b_ref, r_ref, o_ref):
    o_ref[...] = (a_ref[0] + b_ref[0]) * r_ref[...]


def _add2r(p, r, bm=2048):
    _, n, m = p.shape
    return pl.pallas_call(
        _add2r_body,
        grid=(n // bm,),
        in_specs=[pl.BlockSpec((1, bm, m), lambda i: (0, i, 0)),
                  pl.BlockSpec((1, bm, m), lambda i: (1, i, 0)),
                  pl.BlockSpec((bm, 1), lambda i: (i, 0))],
        out_specs=pl.BlockSpec((bm, m), lambda i: (i, 0)),
        out_shape=jax.ShapeDtypeStruct((n, m), jnp.float32),
    )(p, p, r)


@functools.lru_cache(maxsize=None)
def _gat_phase1(dd):
    mesh = plsc.VectorSubcoreMesh(core_axis_name="c", subcore_axis_name="s")

    def body(xl_hbm, xr_hbm, src_hbm, dst_hbm, attb_hbm, mg_hbm, zs_hbm,
             ex_hbm, s_part_hbm,
             attb_v, mg_v, src2d, dst2d, ex2d,
             xl0, xl1, xr0, xr1, s_sh,
             sem_xl0, sem_xl1, sem_xr0, sem_xr1, sem_sc):
        c = lax.axis_index("c")
        s = lax.axis_index("s")
        wid = s * 2 + c
        pltpu.sync_copy(attb_hbm, attb_v)
        pltpu.sync_copy(mg_hbm, mg_v)
        pltpu.sync_copy(src_hbm.at[pl.ds(wid * _NSUB, _NSUB), :], src2d)
        pltpu.sync_copy(dst_hbm.at[pl.ds(wid * _NSUB, _NSUB), :], dst2d)
        pltpu.sync_copy(zs_hbm, s_sh.at[pl.ds(s * 640, 640)])
        plsc.subcore_barrier()
        mgv = mg_v[...]
        iota = lax.iota(jnp.int32, 16)
        xls, xrs = [xl0, xl1], [xr0, xr1]
        sem_xl, sem_xr = [sem_xl0, sem_xl1], [sem_xr0, sem_xr1]

        descs = {}

        def start(j):
            b = j % 2
            descs[j] = (
                pltpu.async_copy(xl_hbm.at[src2d.at[j]], xls[b], sem_xl[b]),
                pltpu.async_copy(xr_hbm.at[dst2d.at[j]], xrs[b], sem_xr[b]),
            )

        start(0)
        start(1)
        sc_descs = {}
        for j in range(_NSUB):
            d1, d2 = descs.pop(j)
            d1.wait()
            d2.wait()
            xlb, xrb = xls[j % 2], xrs[j % 2]

            def group(g, carry2, j=j, xlb=xlb, xrb=xrb):
                rows = iota + g * 16

                def dstep(d, acc):
                    dvec = jnp.full((16,), d, jnp.int32)
                    a = plsc.load_gather(xlb, [rows, dvec])
                    b2 = plsc.load_gather(xrb, [rows, dvec])
                    v = a + b2
                    lr = jnp.maximum(v, 0.2 * v)
                    return acc + lr * attb_v[pl.ds(d * 16, 16)]

                logit = lax.fori_loop(0, dd, dstep,
                                      jnp.zeros((16,), jnp.float32), unroll=8)
                ex2d[j, pl.ds(g * 16, 16)] = jnp.exp(logit - mgv)
                return carry2

            lax.fori_loop(0, _SUB // 16, group, 0)
            if j + 2 < _NSUB:
                start(j + 2)
            sc_descs[j] = pltpu.async_copy(ex2d.at[j], s_sh.at[dst2d.at[j]],
                                           sem_sc, add=True)
            if j >= 8:
                sc_descs.pop(j - 8).wait()
        for dsc in sc_descs.values():
            dsc.wait()
        pltpu.sync_copy(ex2d, ex_hbm.at[pl.ds(wid * _NSUB, _NSUB), :])
        plsc.subcore_barrier()

        @pl.when(s == 0)
        def _():
            pltpu.sync_copy(s_sh, s_part_hbm.at[c])

    return pl.kernel(
        body,
        compiler_params=pltpu.CompilerParams(use_tc_tiling_on_sc=False,
                                             needs_layout_passes=False),
        out_type=[jax.ShapeDtypeStruct((_EP // _SUB, _SUB), jnp.float32),
                  jax.ShapeDtypeStruct((2, _NP), jnp.float32)],
        mesh=mesh,
        scratch_types=[
            pltpu.VMEM((dd * 16,), jnp.float32),
            pltpu.VMEM((16,), jnp.float32),
            pltpu.VMEM((_NSUB, _SUB), jnp.int32),
            pltpu.VMEM((_NSUB, _SUB), jnp.int32),
            pltpu.VMEM((_NSUB, _SUB), jnp.float32),
            pltpu.VMEM((_SUB, dd), jnp.float32),
            pltpu.VMEM((_SUB, dd), jnp.float32),
            pltpu.VMEM((_SUB, dd), jnp.float32),
            pltpu.VMEM((_SUB, dd), jnp.float32),
            pltpu.VMEM_SHARED((_NP,), jnp.float32),
            pltpu.SemaphoreType.DMA,
            pltpu.SemaphoreType.DMA,
            pltpu.SemaphoreType.DMA,
            pltpu.SemaphoreType.DMA,
            pltpu.SemaphoreType.DMA,
        ],
    )


@functools.lru_cache(maxsize=None)
def _gat_phase2(dd):
    mesh = plsc.VectorSubcoreMesh(core_axis_name="c", subcore_axis_name="s")
    nb = 2 if dd == 64 else 1

    def body(xl_hbm, src_hbm, dst_hbm, ex_hbm, zo_hbm, out_part_hbm, *scr):
        if nb == 2:
            (src2d, dst2d, ex2d, xl0, xl1, sc0, sc1, out_sh,
             sem_xl0, sem_xl1, sem_sc0, sem_sc1) = scr
            xls, scs = [xl0, xl1], [sc0, sc1]
            sem_xl, sem_sc = [sem_xl0, sem_xl1], [sem_sc0, sem_sc1]
        else:
            (src2d, dst2d, ex2d, xl0, sc0, out_sh, sem_xl0, sem_sc0) = scr
            xls, scs = [xl0], [sc0]
            sem_xl, sem_sc = [sem_xl0], [sem_sc0]
        c = lax.axis_index("c")
        s = lax.axis_index("s")
        wid = s * 2 + c
        pltpu.sync_copy(src_hbm.at[pl.ds(wid * _NSUB, _NSUB), :], src2d)
        pltpu.sync_copy(dst_hbm.at[pl.ds(wid * _NSUB, _NSUB), :], dst2d)
        pltpu.sync_copy(ex_hbm.at[pl.ds(wid * _NSUB, _NSUB), :], ex2d)
        pltpu.sync_copy(zo_hbm, out_sh.at[pl.ds(s * 640, 640), :])
        plsc.subcore_barrier()
        iota = lax.iota(jnp.int32, 16)

        descs = {}

        def start(j):
            b = j % nb
            descs[j] = pltpu.async_copy(xl_hbm.at[src2d.at[j]], xls[b],
                                        sem_xl[b])

        for t in range(nb):
            start(t)
        sc_descs = {}
        for j in range(_NSUB):
            descs.pop(j).wait()
            b = j % nb
            if (j - nb) in sc_descs:
                sc_descs.pop(j - nb).wait()
            xlb, scb = xls[b], scs[b]

            def group(g, carry2, j=j, xlb=xlb, scb=scb):
                rows = iota + g * 16
                alpha = ex2d[j, pl.ds(g * 16, 16)]

                def dstep(d, carry3):
                    dvec = jnp.full((16,), d, jnp.int32)
                    valv = plsc.load_gather(xlb, [rows, dvec])
                    plsc.store_scatter(scb, [rows, dvec], valv * alpha)
                    return carry3

                lax.fori_loop(0, dd, dstep, 0, unroll=8)
                return carry2

            lax.fori_loop(0, _SUB // 16, group, 0)
            sc_descs[j] = pltpu.async_copy(scb, out_sh.at[dst2d.at[j]],
                                           sem_sc[b], add=True)
            if j + nb < _NSUB:
                start(j + nb)
        for dsc in sc_descs.values():
            dsc.wait()
        plsc.subcore_barrier()
        pltpu.sync_copy(out_sh.at[pl.ds(s * 640, 640), :],
                        out_part_hbm.at[c, pl.ds(s * 640, 640), :])

    scratch = [
        pltpu.VMEM((_NSUB, _SUB), jnp.int32),
        pltpu.VMEM((_NSUB, _SUB), jnp.int32),
        pltpu.VMEM((_NSUB, _SUB), jnp.float32),
    ]
    scratch += [pltpu.VMEM((_SUB, dd), jnp.float32)] * nb
    scratch += [pltpu.VMEM((_SUB, dd), jnp.float32)] * nb
    scratch += [pltpu.VMEM_SHARED((_NP, dd), jnp.float32)]
    scratch += [pltpu.SemaphoreType.DMA] * (2 * nb)
    return pl.kernel(
        body,
        compiler_params=pltpu.CompilerParams(use_tc_tiling_on_sc=False,
                                             needs_layout_passes=False),
        out_type=jax.ShapeDtypeStruct((2, _NP, dd), jnp.float32),
        mesh=mesh,
        scratch_types=scratch,
    )


def _gatv2_sc(xl, xr, mg, att, src2d, dst2d):
    dd = xl.shape[1]
    attb = jnp.repeat(att, 16)
    mg16 = jnp.full((16,), mg, jnp.float32)
    zs = jnp.zeros((640,), jnp.float32)
    zo = jnp.zeros((640, dd), jnp.float32)
    ex, s_part = _gat_phase1(dd)(xl, xr, src2d, dst2d, attb, mg16, zs)
    r = _recip(s_part).reshape(_NP, 1)
    out_part = _gat_phase2(dd)(xl, src2d, dst2d, ex, zo)
    return _add2r(out_part, r)


def kernel(feat, feat_a, graph_neigh, edge_index, adj,
           Wl_zip, Wr_zip, att_zip, Wl_eco, Wr_eco, att_eco, W_disc, b_disc):
    n = feat.shape[0]
    e = edge_index.shape[1]
    srcp = jnp.concatenate(
        [edge_index[0], jnp.full((_EP - e,), _PAD_IDX, jnp.int32)]
    ).reshape(_EP // _SUB, _SUB)
    dstp = jnp.concatenate(
        [edge_index[1], jnp.full((_EP - e,), _PAD_IDX, jnp.int32)]
    ).reshape(_EP // _SUB, _SUB)

    featp = jnp.pad(feat, ((0, _NP - n), (0, 0)))
    featap = jnp.pad(feat_a, ((0, _NP - n), (0, 0)))

    att_nz = jnp.sqrt(jnp.sum(att_zip * att_zip))
    xl1, nl1 = _proj(featp, Wl_zip.T)
    xr1, nr1 = _proj(featp, Wr_zip.T)
    xl2, nl2 = _proj(featap, Wl_zip.T)
    xr2, nr2 = _proj(featap, Wr_zip.T)
    mg1 = att_nz * (jnp.sqrt(jnp.max(nl1)) + jnp.sqrt(jnp.max(nr1)))
    mg2 = att_nz * (jnp.sqrt(jnp.max(nl2)) + jnp.sqrt(jnp.max(nr2)))

    z = _gatv2_sc(xl1, xr1, mg1, att_zip, srcp, dstp)
    z_a = _gatv2_sc(xl2, xr2, mg2, att_zip, srcp, dstp)

    zz = _mm(adj, jnp.concatenate([z[:n], z_a[:n]], axis=1))
    z2 = zz[:, :64]
    z_a2 = zz[:, 64:]

    z2p = jnp.pad(z2, ((0, _NP - n), (0, 0)))
    att_ne = jnp.sqrt(jnp.sum(att_eco * att_eco))
    xl3, nl3 = _proj(z2p, Wl_eco.T)
    xr3, nr3 = _proj(z2p, Wr_eco.T)
    mg3 = att_ne * (jnp.sqrt(jnp.max(nl3)) + jnp.sqrt(jnp.max(nr3)))
    h = _gatv2_sc(xl3, xr3, mg3, att_eco, srcp, dstp)
    h2 = _mm(adj, h[:n])

    emb = jax.nn.relu(z2)
    emb_a = jax.nn.relu(z_a2)

    vs, rs = _mm_rowsum(graph_neigh, jnp.concatenate([emb, emb_a], axis=1))
    row_sum = rs[:, :1]

    def _finish_readout(ge):
        ge = ge / row_sum
        norm = jnp.sqrt(jnp.sum(ge * ge, axis=1, keepdims=True))
        return jax.nn.sigmoid(ge / jnp.maximum(norm, 1e-12))

    g = _finish_readout(vs[:, :64])
    g_a = _finish_readout(vs[:, 64:])

    a1 = emb @ W_disc
    a2 = emb_a @ W_disc
    sc_1 = jnp.sum(a1 * g, axis=1, keepdims=True) + b_disc
    sc_2 = jnp.sum(a2 * g, axis=1, keepdims=True) + b_disc
    sc_1a = jnp.sum(a2 * g_a, axis=1, keepdims=True) + b_disc
    sc_2a = jnp.sum(a1 * g_a, axis=1, keepdims=True) + b_disc
    ret = jax.nn.sigmoid(jnp.concatenate([sc_1, sc_2], axis=1))
    ret_a = jax.nn.sigmoid(jnp.concatenate([sc_1a, sc_2a], axis=1))

    return (z2, h2, ret, ret_a)


# speedup vs baseline: 2.9743x; 2.1501x over previous
import functools

import jax
import jax.numpy as jnp
from jax import lax
from jax.experimental import pallas as pl
from jax.experimental.pallas import tpu as pltpu
from jax.experimental.pallas import tpu_sc as plsc

_N = 10000
_NP = 10240
_EP = 163840
_SUB = 128
_NSUB = 40
_TILE_E = _SUB * _NSUB
_PAD_IDX = 10200


def _mm_body(a_ref, b_ref, o_ref):
    o_ref[...] = jnp.dot(a_ref[...], b_ref[...],
                         preferred_element_type=jnp.float32)


def _mm(a, b, bm=400):
    n, kdim = a.shape
    m = b.shape[1]
    return pl.pallas_call(
        _mm_body,
        grid=(n // bm,),
        in_specs=[pl.BlockSpec((bm, kdim), lambda i: (i, 0)),
                  pl.BlockSpec((kdim, m), lambda i: (0, 0))],
        out_specs=pl.BlockSpec((bm, m), lambda i: (i, 0)),
        out_shape=jax.ShapeDtypeStruct((n, m), jnp.float32),
    )(a, b)


def _mm_rowsum_body(a_ref, b_ref, o_ref, rs_ref):
    a = a_ref[...]
    o_ref[...] = jnp.dot(a, b_ref[...], preferred_element_type=jnp.float32)
    rs_ref[...] = jnp.broadcast_to(jnp.sum(a, axis=1, keepdims=True),
                                   rs_ref.shape)


def _mm_rowsum(a, b, bm=400):
    n, kdim = a.shape
    m = b.shape[1]
    return pl.pallas_call(
        _mm_rowsum_body,
        grid=(n // bm,),
        in_specs=[pl.BlockSpec((bm, kdim), lambda i: (i, 0)),
                  pl.BlockSpec((kdim, m), lambda i: (0, 0))],
        out_specs=[pl.BlockSpec((bm, m), lambda i: (i, 0)),
                   pl.BlockSpec((bm, 8), lambda i: (i, 0))],
        out_shape=[jax.ShapeDtypeStruct((n, m), jnp.float32),
                   jax.ShapeDtypeStruct((n, 8), jnp.float32)],
    )(a, b)


def _proj_body(x_ref, w_ref, o_ref, nm_ref):
    i = pl.program_id(0)
    y = jnp.dot(x_ref[...], w_ref[...], preferred_element_type=jnp.float32)
    o_ref[...] = y
    blk = jnp.full(nm_ref.shape, jnp.max(jnp.sum(y * y, axis=1)))

    @pl.when(i == 0)
    def _():
        nm_ref[...] = blk

    @pl.when(i != 0)
    def _():
        nm_ref[...] = jnp.maximum(nm_ref[...], blk)


def _proj(x, wt, bm=512):
    n, kdim = x.shape
    m = wt.shape[1]
    return pl.pallas_call(
        _proj_body,
        grid=(n // bm,),
        in_specs=[pl.BlockSpec((bm, kdim), lambda i: (i, 0)),
                  pl.BlockSpec((kdim, m), lambda i: (0, 0))],
        out_specs=[pl.BlockSpec((bm, m), lambda i: (i, 0)),
                   pl.BlockSpec((8, 128), lambda i: (0, 0))],
        out_shape=[jax.ShapeDtypeStruct((n, m), jnp.float32),
                   jax.ShapeDtypeStruct((8, 128), jnp.float32)],
    )(x, wt)


def _recip_body(s_ref, o_ref):
    o_ref[...] = 1.0 / (s_ref[0:1, :] + s_ref[1:2, :] + 1e-16)


def _recip(s_part):
    out = pl.pallas_call(
        _recip_body,
        out_shape=jax.ShapeDtypeStruct((1, _NP), jnp.float32),
    )(s_part)
    return out.reshape(_NP)


def _add2r_body(a_ref, b_ref, r_ref, o_ref):
    o_ref[...] = (a_ref[0] + b_ref[0]) * r_ref[...]


def _add2r(p, r, bm=2048):
    _, n, m = p.shape
    return pl.pallas_call(
        _add2r_body,
        grid=(n // bm,),
        in_specs=[pl.BlockSpec((1, bm, m), lambda i: (0, i, 0)),
                  pl.BlockSpec((1, bm, m), lambda i: (1, i, 0)),
                  pl.BlockSpec((bm, 1), lambda i: (i, 0))],
        out_specs=pl.BlockSpec((bm, m), lambda i: (i, 0)),
        out_shape=jax.ShapeDtypeStruct((n, m), jnp.float32),
    )(p, p, r)


@functools.lru_cache(maxsize=None)
def _gat_phase1(dd):
    mesh = plsc.VectorSubcoreMesh(core_axis_name="c", subcore_axis_name="s")

    def body(xl_hbm, xr_hbm, src_hbm, dst_hbm, attb_hbm, mg_hbm, zs_hbm,
             ex_hbm, s_part_hbm,
             attb_v, mg_v, src2d, dst2d, ex2d,
             xl0, xl1, xr0, xr1, s_sh,
             sem_xl0, sem_xl1, sem_xr0, sem_xr1, sem_sc):
        c = lax.axis_index("c")
        s = lax.axis_index("s")
        wid = s * 2 + c
        pltpu.sync_copy(attb_hbm, attb_v)
        pltpu.sync_copy(mg_hbm, mg_v)
        pltpu.sync_copy(src_hbm.at[pl.ds(wid * _NSUB, _NSUB), :], src2d)
        pltpu.sync_copy(dst_hbm.at[pl.ds(wid * _NSUB, _NSUB), :], dst2d)
        pltpu.sync_copy(zs_hbm, s_sh.at[pl.ds(s * 640, 640)])
        plsc.subcore_barrier()
        mgv = mg_v[...]
        iota = lax.iota(jnp.int32, 16)
        xls, xrs = [xl0, xl1], [xr0, xr1]
        sem_xl, sem_xr = [sem_xl0, sem_xl1], [sem_xr0, sem_xr1]

        descs = {}

        def start(j):
            b = j % 2
            descs[j] = (
                pltpu.async_copy(xl_hbm.at[src2d.at[j]], xls[b], sem_xl[b]),
                pltpu.async_copy(xr_hbm.at[dst2d.at[j]], xrs[b], sem_xr[b]),
            )

        start(0)
        start(1)
        sc_descs = {}
        for j in range(_NSUB):
            d1, d2 = descs.pop(j)
            d1.wait()
            d2.wait()
            xlb, xrb = xls[j % 2], xrs[j % 2]

            def group(g, carry2, j=j, xlb=xlb, xrb=xrb):
                rows = iota + g * 16

                def dstep(d, acc):
                    dvec = (d + iota) & (dd - 1)
                    a = plsc.load_gather(xlb, [rows, dvec])
                    b2 = plsc.load_gather(xrb, [rows, dvec])
                    v = a + b2
                    lr = jnp.maximum(v, 0.2 * v)
                    return acc + lr * attb_v[pl.ds(d, 16)]

                logit = lax.fori_loop(0, dd, dstep,
                                      jnp.zeros((16,), jnp.float32), unroll=8)
                ex2d[j, pl.ds(g * 16, 16)] = jnp.exp(logit - mgv)
                return carry2

            lax.fori_loop(0, _SUB // 16, group, 0)
            if j + 2 < _NSUB:
                start(j + 2)
            sc_descs[j] = pltpu.async_copy(ex2d.at[j], s_sh.at[dst2d.at[j]],
                                           sem_sc, add=True)
            if j >= 8:
                sc_descs.pop(j - 8).wait()
        for dsc in sc_descs.values():
            dsc.wait()
        pltpu.sync_copy(ex2d, ex_hbm.at[pl.ds(wid * _NSUB, _NSUB), :])
        plsc.subcore_barrier()

        @pl.when(s == 0)
        def _():
            pltpu.sync_copy(s_sh, s_part_hbm.at[c])

    return pl.kernel(
        body,
        compiler_params=pltpu.CompilerParams(use_tc_tiling_on_sc=False,
                                             needs_layout_passes=False),
        out_type=[jax.ShapeDtypeStruct((_EP // _SUB, _SUB), jnp.float32),
                  jax.ShapeDtypeStruct((2, _NP), jnp.float32)],
        mesh=mesh,
        scratch_types=[
            pltpu.VMEM((dd + 16,), jnp.float32),
            pltpu.VMEM((16,), jnp.float32),
            pltpu.VMEM((_NSUB, _SUB), jnp.int32),
            pltpu.VMEM((_NSUB, _SUB), jnp.int32),
            pltpu.VMEM((_NSUB, _SUB), jnp.float32),
            pltpu.VMEM((_SUB, dd), jnp.float32),
            pltpu.VMEM((_SUB, dd), jnp.float32),
            pltpu.VMEM((_SUB, dd), jnp.float32),
            pltpu.VMEM((_SUB, dd), jnp.float32),
            pltpu.VMEM_SHARED((_NP,), jnp.float32),
            pltpu.SemaphoreType.DMA,
            pltpu.SemaphoreType.DMA,
            pltpu.SemaphoreType.DMA,
            pltpu.SemaphoreType.DMA,
            pltpu.SemaphoreType.DMA,
        ],
    )


@functools.lru_cache(maxsize=None)
def _gat_phase2(dd):
    mesh = plsc.VectorSubcoreMesh(core_axis_name="c", subcore_axis_name="s")
    nb = 2 if dd == 64 else 1

    def body(xl_hbm, src_hbm, dst_hbm, ex_hbm, zo_hbm, out_part_hbm, *scr):
        if nb == 2:
            (src2d, dst2d, ex2d, xl0, xl1, sc0, sc1, out_sh,
             sem_xl0, sem_xl1, sem_sc0, sem_sc1) = scr
            xls, scs = [xl0, xl1], [sc0, sc1]
            sem_xl, sem_sc = [sem_xl0, sem_xl1], [sem_sc0, sem_sc1]
        else:
            (src2d, dst2d, ex2d, xl0, sc0, out_sh, sem_xl0, sem_sc0) = scr
            xls, scs = [xl0], [sc0]
            sem_xl, sem_sc = [sem_xl0], [sem_sc0]
        c = lax.axis_index("c")
        s = lax.axis_index("s")
        wid = s * 2 + c
        pltpu.sync_copy(src_hbm.at[pl.ds(wid * _NSUB, _NSUB), :], src2d)
        pltpu.sync_copy(dst_hbm.at[pl.ds(wid * _NSUB, _NSUB), :], dst2d)
        pltpu.sync_copy(ex_hbm.at[pl.ds(wid * _NSUB, _NSUB), :], ex2d)
        pltpu.sync_copy(zo_hbm, out_sh.at[pl.ds(s * 640, 640), :])
        plsc.subcore_barrier()
        iota = lax.iota(jnp.int32, 16)

        descs = {}

        def start(j):
            b = j % nb
            descs[j] = pltpu.async_copy(xl_hbm.at[src2d.at[j]], xls[b],
                                        sem_xl[b])

        for t in range(nb):
            start(t)
        sc_descs = {}
        for j in range(_NSUB):
            descs.pop(j).wait()
            b = j % nb
            if (j - nb) in sc_descs:
                sc_descs.pop(j - nb).wait()
            xlb, scb = xls[b], scs[b]

            def group(g, carry2, j=j, xlb=xlb, scb=scb):
                rows = iota + g * 16
                alpha = ex2d[j, pl.ds(g * 16, 16)]

                def dstep(d, carry3):
                    dvec = (d + iota) & (dd - 1)
                    valv = plsc.load_gather(xlb, [rows, dvec])
                    plsc.store_scatter(scb, [rows, dvec], valv * alpha)
                    return carry3

                lax.fori_loop(0, dd, dstep, 0, unroll=8)
                return carry2

            lax.fori_loop(0, _SUB // 16, group, 0)
            sc_descs[j] = pltpu.async_copy(scb, out_sh.at[dst2d.at[j]],
                                           sem_sc[b], add=True)
            if j + nb < _NSUB:
                start(j + nb)
        for dsc in sc_descs.values():
            dsc.wait()
        plsc.subcore_barrier()
        pltpu.sync_copy(out_sh.at[pl.ds(s * 640, 640), :],
                        out_part_hbm.at[c, pl.ds(s * 640, 640), :])

    scratch = [
        pltpu.VMEM((_NSUB, _SUB), jnp.int32),
        pltpu.VMEM((_NSUB, _SUB), jnp.int32),
        pltpu.VMEM((_NSUB, _SUB), jnp.float32),
    ]
    scratch += [pltpu.VMEM((_SUB, dd), jnp.float32)] * nb
    scratch += [pltpu.VMEM((_SUB, dd), jnp.float32)] * nb
    scratch += [pltpu.VMEM_SHARED((_NP, dd), jnp.float32)]
    scratch += [pltpu.SemaphoreType.DMA] * (2 * nb)
    return pl.kernel(
        body,
        compiler_params=pltpu.CompilerParams(use_tc_tiling_on_sc=False,
                                             needs_layout_passes=False),
        out_type=jax.ShapeDtypeStruct((2, _NP, dd), jnp.float32),
        mesh=mesh,
        scratch_types=scratch,
    )


def _gatv2_sc(xl, xr, mg, att, src2d, dst2d):
    dd = xl.shape[1]
    attb = jnp.concatenate([att, att[:16]])
    mg16 = jnp.full((16,), mg, jnp.float32)
    zs = jnp.zeros((640,), jnp.float32)
    zo = jnp.zeros((640, dd), jnp.float32)
    ex, s_part = _gat_phase1(dd)(xl, xr, src2d, dst2d, attb, mg16, zs)
    r = _recip(s_part).reshape(_NP, 1)
    out_part = _gat_phase2(dd)(xl, src2d, dst2d, ex, zo)
    return _add2r(out_part, r)


def kernel(feat, feat_a, graph_neigh, edge_index, adj,
           Wl_zip, Wr_zip, att_zip, Wl_eco, Wr_eco, att_eco, W_disc, b_disc):
    n = feat.shape[0]
    e = edge_index.shape[1]
    srcp = jnp.concatenate(
        [edge_index[0], jnp.full((_EP - e,), _PAD_IDX, jnp.int32)]
    ).reshape(_EP // _SUB, _SUB)
    dstp = jnp.concatenate(
        [edge_index[1], jnp.full((_EP - e,), _PAD_IDX, jnp.int32)]
    ).reshape(_EP // _SUB, _SUB)

    featp = jnp.pad(feat, ((0, _NP - n), (0, 0)))
    featap = jnp.pad(feat_a, ((0, _NP - n), (0, 0)))

    att_nz = jnp.sqrt(jnp.sum(att_zip * att_zip))
    xl1, nl1 = _proj(featp, Wl_zip.T)
    xr1, nr1 = _proj(featp, Wr_zip.T)
    xl2, nl2 = _proj(featap, Wl_zip.T)
    xr2, nr2 = _proj(featap, Wr_zip.T)
    mg1 = att_nz * (jnp.sqrt(jnp.max(nl1)) + jnp.sqrt(jnp.max(nr1)))
    mg2 = att_nz * (jnp.sqrt(jnp.max(nl2)) + jnp.sqrt(jnp.max(nr2)))

    z = _gatv2_sc(xl1, xr1, mg1, att_zip, srcp, dstp)
    z_a = _gatv2_sc(xl2, xr2, mg2, att_zip, srcp, dstp)

    zz = _mm(adj, jnp.concatenate([z[:n], z_a[:n]], axis=1))
    z2 = zz[:, :64]
    z_a2 = zz[:, 64:]

    z2p = jnp.pad(z2, ((0, _NP - n), (0, 0)))
    att_ne = jnp.sqrt(jnp.sum(att_eco * att_eco))
    xl3, nl3 = _proj(z2p, Wl_eco.T)
    xr3, nr3 = _proj(z2p, Wr_eco.T)
    mg3 = att_ne * (jnp.sqrt(jnp.max(nl3)) + jnp.sqrt(jnp.max(nr3)))
    h = _gatv2_sc(xl3, xr3, mg3, att_eco, srcp, dstp)
    h2 = _mm(adj, h[:n])

    emb = jax.nn.relu(z2)
    emb_a = jax.nn.relu(z_a2)

    vs, rs = _mm_rowsum(graph_neigh, jnp.concatenate([emb, emb_a], axis=1))
    row_sum = rs[:, :1]

    def _finish_readout(ge):
        ge = ge / row_sum
        norm = jnp.sqrt(jnp.sum(ge * ge, axis=1, keepdims=True))
        return jax.nn.sigmoid(ge / jnp.maximum(norm, 1e-12))

    g = _finish_readout(vs[:, :64])
    g_a = _finish_readout(vs[:, 64:])

    a1 = emb @ W_disc
    a2 = emb_a @ W_disc
    sc_1 = jnp.sum(a1 * g, axis=1, keepdims=True) + b_disc
    sc_2 = jnp.sum(a2 * g, axis=1, keepdims=True) + b_disc
    sc_1a = jnp.sum(a2 * g_a, axis=1, keepdims=True) + b_disc
    sc_2a = jnp.sum(a1 * g_a, axis=1, keepdims=True) + b_disc
    ret = jax.nn.sigmoid(jnp.concatenate([sc_1, sc_2], axis=1))
    ret_a = jax.nn.sigmoid(jnp.concatenate([sc_1a, sc_2a], axis=1))

    return (z2, h2, ret, ret_a)


# speedup vs baseline: 3.0389x; 1.0217x over previous
import functools

import jax
import jax.numpy as jnp
from jax import lax
from jax.experimental import pallas as pl
from jax.experimental.pallas import tpu as pltpu
from jax.experimental.pallas import tpu_sc as plsc

_N = 10000
_NP = 10240
_EP = 163840
_SUB = 128
_NSUB = 40
_TILE_E = _SUB * _NSUB
_PAD_IDX = 10200


def _mm_body(a_ref, b_ref, o_ref):
    o_ref[...] = jnp.dot(a_ref[...], b_ref[...],
                         preferred_element_type=jnp.float32)


def _mm(a, b, bm=400):
    n, kdim = a.shape
    m = b.shape[1]
    return pl.pallas_call(
        _mm_body,
        grid=(n // bm,),
        in_specs=[pl.BlockSpec((bm, kdim), lambda i: (i, 0)),
                  pl.BlockSpec((kdim, m), lambda i: (0, 0))],
        out_specs=pl.BlockSpec((bm, m), lambda i: (i, 0)),
        out_shape=jax.ShapeDtypeStruct((n, m), jnp.float32),
    )(a, b)


def _mm_rowsum_body(a_ref, b_ref, o_ref, rs_ref):
    a = a_ref[...]
    o_ref[...] = jnp.dot(a, b_ref[...], preferred_element_type=jnp.float32)
    rs_ref[...] = jnp.broadcast_to(jnp.sum(a, axis=1, keepdims=True),
                                   rs_ref.shape)


def _mm_rowsum(a, b, bm=400):
    n, kdim = a.shape
    m = b.shape[1]
    return pl.pallas_call(
        _mm_rowsum_body,
        grid=(n // bm,),
        in_specs=[pl.BlockSpec((bm, kdim), lambda i: (i, 0)),
                  pl.BlockSpec((kdim, m), lambda i: (0, 0))],
        out_specs=[pl.BlockSpec((bm, m), lambda i: (i, 0)),
                   pl.BlockSpec((bm, 8), lambda i: (i, 0))],
        out_shape=[jax.ShapeDtypeStruct((n, m), jnp.float32),
                   jax.ShapeDtypeStruct((n, 8), jnp.float32)],
    )(a, b)


def _proj_body(x_ref, w_ref, o_ref, nm_ref):
    i = pl.program_id(0)
    y = jnp.dot(x_ref[...], w_ref[...], preferred_element_type=jnp.float32)
    o_ref[...] = y
    blk = jnp.full(nm_ref.shape, jnp.max(jnp.sum(y * y, axis=1)))

    @pl.when(i == 0)
    def _():
        nm_ref[...] = blk

    @pl.when(i != 0)
    def _():
        nm_ref[...] = jnp.maximum(nm_ref[...], blk)


def _proj(x, wt, bm=512):
    n, kdim = x.shape
    m = wt.shape[1]
    return pl.pallas_call(
        _proj_body,
        grid=(n // bm,),
        in_specs=[pl.BlockSpec((bm, kdim), lambda i: (i, 0)),
                  pl.BlockSpec((kdim, m), lambda i: (0, 0))],
        out_specs=[pl.BlockSpec((bm, m), lambda i: (i, 0)),
                   pl.BlockSpec((8, 128), lambda i: (0, 0))],
        out_shape=[jax.ShapeDtypeStruct((n, m), jnp.float32),
                   jax.ShapeDtypeStruct((8, 128), jnp.float32)],
    )(x, wt)


def _recip_body(s_ref, o_ref):
    o_ref[...] = 1.0 / (s_ref[0:1, :] + s_ref[1:2, :] + 1e-16)


def _recip(s_part):
    out = pl.pallas_call(
        _recip_body,
        out_shape=jax.ShapeDtypeStruct((1, _NP), jnp.float32),
    )(s_part)
    return out.reshape(_NP)


def _add2r_body(a_ref, b_ref, r_ref, o_ref):
    o_ref[...] = (a_ref[0] + b_ref[0]) * r_ref[...]


def _add2r(p, r, bm=2048):
    _, n, m = p.shape
    return pl.pallas_call(
        _add2r_body,
        grid=(n // bm,),
        in_specs=[pl.BlockSpec((1, bm, m), lambda i: (0, i, 0)),
                  pl.BlockSpec((1, bm, m), lambda i: (1, i, 0)),
                  pl.BlockSpec((bm, 1), lambda i: (i, 0))],
        out_specs=pl.BlockSpec((bm, m), lambda i: (i, 0)),
        out_shape=jax.ShapeDtypeStruct((n, m), jnp.float32),
    )(p, p, r)


@functools.lru_cache(maxsize=None)
def _gat_phase1(dd):
    mesh = plsc.VectorSubcoreMesh(core_axis_name="c", subcore_axis_name="s")

    def body(xl_hbm, xr_hbm, src_hbm, dst_hbm, attb_hbm, mg_hbm, zs_hbm,
             ex_hbm, s_part_hbm,
             attb_v, mg_v, src2d, dst2d, ex2d,
             xl0, xl1, xr0, xr1, s_sh,
             sem_xl0, sem_xl1, sem_xr0, sem_xr1, sem_sc):
        c = lax.axis_index("c")
        s = lax.axis_index("s")
        wid = s * 2 + c
        pltpu.sync_copy(attb_hbm, attb_v)
        pltpu.sync_copy(mg_hbm, mg_v)
        pltpu.sync_copy(src_hbm.at[pl.ds(wid * _NSUB, _NSUB), :], src2d)
        pltpu.sync_copy(dst_hbm.at[pl.ds(wid * _NSUB, _NSUB), :], dst2d)
        pltpu.sync_copy(zs_hbm, s_sh.at[pl.ds(s * 640, 640)])
        plsc.subcore_barrier()
        mgv = mg_v[...]
        iota = lax.iota(jnp.int32, 16)
        xls, xrs = [xl0, xl1], [xr0, xr1]
        sem_xl, sem_xr = [sem_xl0, sem_xl1], [sem_xr0, sem_xr1]

        descs = {}

        def start(j):
            b = j % 2
            descs[j] = (
                pltpu.async_copy(xl_hbm.at[src2d.at[j]], xls[b], sem_xl[b]),
                pltpu.async_copy(xr_hbm.at[dst2d.at[j]], xrs[b], sem_xr[b]),
            )

        start(0)
        start(1)
        sc_descs = {}
        for j in range(_NSUB):
            d1, d2 = descs.pop(j)
            d1.wait()
            d2.wait()
            xlb, xrb = xls[j % 2], xrs[j % 2]

            def group(g, carry2, j=j, xlb=xlb, xrb=xrb):
                rows = iota + g * 16

                def dstep(d, acc):
                    dvec = (d + iota) & (dd - 1)
                    a = plsc.load_gather(xlb, [rows, dvec])
                    b2 = plsc.load_gather(xrb, [rows, dvec])
                    v = a + b2
                    lr = jnp.maximum(v, 0.2 * v)
                    return acc + lr * attb_v[pl.ds(d, 16)]

                logit = lax.fori_loop(0, dd, dstep,
                                      jnp.zeros((16,), jnp.float32), unroll=8)
                ex2d[j, pl.ds(g * 16, 16)] = jnp.exp(logit - mgv)
                return carry2

            lax.fori_loop(0, _SUB // 16, group, 0)
            if j + 2 < _NSUB:
                start(j + 2)
            sc_descs[j] = pltpu.async_copy(ex2d.at[j], s_sh.at[dst2d.at[j]],
                                           sem_sc, add=True)
            if j >= 8:
                sc_descs.pop(j - 8).wait()
        for dsc in sc_descs.values():
            dsc.wait()
        pltpu.sync_copy(ex2d, ex_hbm.at[pl.ds(wid * _NSUB, _NSUB), :])
        plsc.subcore_barrier()

        @pl.when(s == 0)
        def _():
            pltpu.sync_copy(s_sh, s_part_hbm.at[c])

    return pl.kernel(
        body,
        compiler_params=pltpu.CompilerParams(use_tc_tiling_on_sc=False,
                                             needs_layout_passes=False),
        out_type=[jax.ShapeDtypeStruct((_EP // _SUB, _SUB), jnp.float32),
                  jax.ShapeDtypeStruct((2, _NP), jnp.float32)],
        mesh=mesh,
        scratch_types=[
            pltpu.VMEM((dd + 16,), jnp.float32),
            pltpu.VMEM((16,), jnp.float32),
            pltpu.VMEM((_NSUB, _SUB), jnp.int32),
            pltpu.VMEM((_NSUB, _SUB), jnp.int32),
            pltpu.VMEM((_NSUB, _SUB), jnp.float32),
            pltpu.VMEM((_SUB, dd), jnp.float32),
            pltpu.VMEM((_SUB, dd), jnp.float32),
            pltpu.VMEM((_SUB, dd), jnp.float32),
            pltpu.VMEM((_SUB, dd), jnp.float32),
            pltpu.VMEM_SHARED((_NP,), jnp.float32),
            pltpu.SemaphoreType.DMA,
            pltpu.SemaphoreType.DMA,
            pltpu.SemaphoreType.DMA,
            pltpu.SemaphoreType.DMA,
            pltpu.SemaphoreType.DMA,
        ],
    )


@functools.lru_cache(maxsize=None)
def _gat_phase2(dd):
    mesh = plsc.VectorSubcoreMesh(core_axis_name="c", subcore_axis_name="s")
    nb = 2 if dd == 64 else 1

    def body(xl_hbm, src_hbm, dst_hbm, ex_hbm, zo_hbm, out_part_hbm, *scr):
        if nb == 2:
            (src2d, dst2d, ex2d, xl0, xl1, sc0, sc1, out_sh,
             sem_xl0, sem_xl1, sem_sc0, sem_sc1) = scr
            xls, scs = [xl0, xl1], [sc0, sc1]
            sem_xl, sem_sc = [sem_xl0, sem_xl1], [sem_sc0, sem_sc1]
        else:
            (src2d, dst2d, ex2d, xl0, sc0, out_sh, sem_xl0, sem_sc0) = scr
            xls, scs = [xl0], [sc0]
            sem_xl, sem_sc = [sem_xl0], [sem_sc0]
        c = lax.axis_index("c")
        s = lax.axis_index("s")
        wid = s * 2 + c
        pltpu.sync_copy(src_hbm.at[pl.ds(wid * _NSUB, _NSUB), :], src2d)
        pltpu.sync_copy(dst_hbm.at[pl.ds(wid * _NSUB, _NSUB), :], dst2d)
        pltpu.sync_copy(ex_hbm.at[pl.ds(wid * _NSUB, _NSUB), :], ex2d)
        pltpu.sync_copy(zo_hbm, out_sh.at[pl.ds(s * 640, 640), :])
        plsc.subcore_barrier()
        iota = lax.iota(jnp.int32, 16)

        descs = {}

        def start(j):
            b = j % nb
            descs[j] = pltpu.async_copy(xl_hbm.at[src2d.at[j]], xls[b],
                                        sem_xl[b])

        for t in range(nb):
            start(t)
        sc_descs = {}
        for j in range(_NSUB):
            descs.pop(j).wait()
            b = j % nb
            if (j - nb) in sc_descs:
                sc_descs.pop(j - nb).wait()
            xlb, scb = xls[b], scs[b]

            def group(g, carry2, j=j, xlb=xlb, scb=scb):
                rows = iota + g * 16
                alpha = ex2d[j, pl.ds(g * 16, 16)]

                def dstep(d, carry3):
                    dvec = (d + iota) & (dd - 1)
                    valv = plsc.load_gather(xlb, [rows, dvec])
                    plsc.store_scatter(scb, [rows, dvec], valv * alpha)
                    return carry3

                lax.fori_loop(0, dd, dstep, 0, unroll=8)
                return carry2

            lax.fori_loop(0, _SUB // 16, group, 0)
            sc_descs[j] = pltpu.async_copy(scb, out_sh.at[dst2d.at[j]],
                                           sem_sc[b], add=True)
            if j + nb < _NSUB:
                start(j + nb)
        for dsc in sc_descs.values():
            dsc.wait()
        plsc.subcore_barrier()
        pltpu.sync_copy(out_sh.at[pl.ds(s * 640, 640), :],
                        out_part_hbm.at[c, pl.ds(s * 640, 640), :])

    scratch = [
        pltpu.VMEM((_NSUB, _SUB), jnp.int32),
        pltpu.VMEM((_NSUB, _SUB), jnp.int32),
        pltpu.VMEM((_NSUB, _SUB), jnp.float32),
    ]
    scratch += [pltpu.VMEM((_SUB, dd), jnp.float32)] * nb
    scratch += [pltpu.VMEM((_SUB, dd), jnp.float32)] * nb
    scratch += [pltpu.VMEM_SHARED((_NP, dd), jnp.float32)]
    scratch += [pltpu.SemaphoreType.DMA] * (2 * nb)
    return pl.kernel(
        body,
        compiler_params=pltpu.CompilerParams(use_tc_tiling_on_sc=False,
                                             needs_layout_passes=False),
        out_type=jax.ShapeDtypeStruct((2, _NP, dd), jnp.float32),
        mesh=mesh,
        scratch_types=scratch,
    )


def _gatv2_sc(xl, xr, mg, att, src2d, dst2d):
    dd = xl.shape[1]
    attb = jnp.concatenate([att, att[:16]])
    mg16 = jnp.full((16,), mg, jnp.float32)
    zs = jnp.zeros((640,), jnp.float32)
    zo = jnp.zeros((640, 64), jnp.float32)
    ex, s_part = _gat_phase1(dd)(xl, xr, src2d, dst2d, attb, mg16, zs)
    r = _recip(s_part).reshape(_NP, 1)
    outs = []
    for lo in range(0, dd, 64):
        part = _gat_phase2(64)(xl[:, lo:lo + 64] + 0.0,
                               src2d, dst2d, ex, zo)
        outs.append(_add2r(part, r))
    if len(outs) == 1:
        return outs[0]
    return jnp.concatenate(outs, axis=1)


def kernel(feat, feat_a, graph_neigh, edge_index, adj,
           Wl_zip, Wr_zip, att_zip, Wl_eco, Wr_eco, att_eco, W_disc, b_disc):
    n = feat.shape[0]
    e = edge_index.shape[1]
    srcp = jnp.concatenate(
        [edge_index[0], jnp.full((_EP - e,), _PAD_IDX, jnp.int32)]
    ).reshape(_EP // _SUB, _SUB)
    dstp = jnp.concatenate(
        [edge_index[1], jnp.full((_EP - e,), _PAD_IDX, jnp.int32)]
    ).reshape(_EP // _SUB, _SUB)

    featp = jnp.pad(feat, ((0, _NP - n), (0, 0)))
    featap = jnp.pad(feat_a, ((0, _NP - n), (0, 0)))

    att_nz = jnp.sqrt(jnp.sum(att_zip * att_zip))
    xl1, nl1 = _proj(featp, Wl_zip.T)
    xr1, nr1 = _proj(featp, Wr_zip.T)
    xl2, nl2 = _proj(featap, Wl_zip.T)
    xr2, nr2 = _proj(featap, Wr_zip.T)
    mg1 = att_nz * (jnp.sqrt(jnp.max(nl1)) + jnp.sqrt(jnp.max(nr1)))
    mg2 = att_nz * (jnp.sqrt(jnp.max(nl2)) + jnp.sqrt(jnp.max(nr2)))

    z = _gatv2_sc(xl1, xr1, mg1, att_zip, srcp, dstp)
    z_a = _gatv2_sc(xl2, xr2, mg2, att_zip, srcp, dstp)

    zz = _mm(adj, jnp.concatenate([z[:n], z_a[:n]], axis=1))
    z2 = zz[:, :64]
    z_a2 = zz[:, 64:]

    z2p = jnp.pad(z2, ((0, _NP - n), (0, 0)))
    att_ne = jnp.sqrt(jnp.sum(att_eco * att_eco))
    xl3, nl3 = _proj(z2p, Wl_eco.T)
    xr3, nr3 = _proj(z2p, Wr_eco.T)
    mg3 = att_ne * (jnp.sqrt(jnp.max(nl3)) + jnp.sqrt(jnp.max(nr3)))
    h = _gatv2_sc(xl3, xr3, mg3, att_eco, srcp, dstp)
    h2 = _mm(adj, h[:n])

    emb = jax.nn.relu(z2)
    emb_a = jax.nn.relu(z_a2)

    vs, rs = _mm_rowsum(graph_neigh, jnp.concatenate([emb, emb_a], axis=1))
    row_sum = rs[:, :1]

    def _finish_readout(ge):
        ge = ge / row_sum
        norm = jnp.sqrt(jnp.sum(ge * ge, axis=1, keepdims=True))
        return jax.nn.sigmoid(ge / jnp.maximum(norm, 1e-12))

    g = _finish_readout(vs[:, :64])
    g_a = _finish_readout(vs[:, 64:])

    a1 = emb @ W_disc
    a2 = emb_a @ W_disc
    sc_1 = jnp.sum(a1 * g, axis=1, keepdims=True) + b_disc
    sc_2 = jnp.sum(a2 * g, axis=1, keepdims=True) + b_disc
    sc_1a = jnp.sum(a2 * g_a, axis=1, keepdims=True) + b_disc
    sc_2a = jnp.sum(a1 * g_a, axis=1, keepdims=True) + b_disc
    ret = jax.nn.sigmoid(jnp.concatenate([sc_1, sc_2], axis=1))
    ret_a = jax.nn.sigmoid(jnp.concatenate([sc_1a, sc_2a], axis=1))

    return (z2, h2, ret, ret_a)


# speedup vs baseline: 3.0532x; 1.0047x over previous
import functools

import jax
import jax.numpy as jnp
from jax import lax
from jax.experimental import pallas as pl
from jax.experimental.pallas import tpu as pltpu
from jax.experimental.pallas import tpu_sc as plsc

_N = 10000
_NP = 10240
_EP = 163840
_SUB = 128
_NSUB = 40
_TILE_E = _SUB * _NSUB
_PAD_IDX = 10200


def _mm_body(a_ref, b_ref, o_ref):
    o_ref[...] = jnp.dot(a_ref[...], b_ref[...],
                         preferred_element_type=jnp.float32)


def _mm(a, b, bm=400):
    n, kdim = a.shape
    m = b.shape[1]
    return pl.pallas_call(
        _mm_body,
        grid=(n // bm,),
        in_specs=[pl.BlockSpec((bm, kdim), lambda i: (i, 0)),
                  pl.BlockSpec((kdim, m), lambda i: (0, 0))],
        out_specs=pl.BlockSpec((bm, m), lambda i: (i, 0)),
        out_shape=jax.ShapeDtypeStruct((n, m), jnp.float32),
    )(a, b)


def _mm_rowsum_body(a_ref, b_ref, o_ref, rs_ref):
    a = a_ref[...]
    o_ref[...] = jnp.dot(a, b_ref[...], preferred_element_type=jnp.float32)
    rs_ref[...] = jnp.broadcast_to(jnp.sum(a, axis=1, keepdims=True),
                                   rs_ref.shape)


def _mm_rowsum(a, b, bm=400):
    n, kdim = a.shape
    m = b.shape[1]
    return pl.pallas_call(
        _mm_rowsum_body,
        grid=(n // bm,),
        in_specs=[pl.BlockSpec((bm, kdim), lambda i: (i, 0)),
                  pl.BlockSpec((kdim, m), lambda i: (0, 0))],
        out_specs=[pl.BlockSpec((bm, m), lambda i: (i, 0)),
                   pl.BlockSpec((bm, 8), lambda i: (i, 0))],
        out_shape=[jax.ShapeDtypeStruct((n, m), jnp.float32),
                   jax.ShapeDtypeStruct((n, 8), jnp.float32)],
    )(a, b)


def _proj_body(x_ref, w_ref, o_ref, nm_ref):
    i = pl.program_id(0)
    y = jnp.dot(x_ref[...], w_ref[...], preferred_element_type=jnp.float32)
    o_ref[...] = y
    blk = jnp.full(nm_ref.shape, jnp.max(jnp.sum(y * y, axis=1)))

    @pl.when(i == 0)
    def _():
        nm_ref[...] = blk

    @pl.when(i != 0)
    def _():
        nm_ref[...] = jnp.maximum(nm_ref[...], blk)


def _proj(x, wt, bm=512):
    n, kdim = x.shape
    m = wt.shape[1]
    return pl.pallas_call(
        _proj_body,
        grid=(n // bm,),
        in_specs=[pl.BlockSpec((bm, kdim), lambda i: (i, 0)),
                  pl.BlockSpec((kdim, m), lambda i: (0, 0))],
        out_specs=[pl.BlockSpec((bm, m), lambda i: (i, 0)),
                   pl.BlockSpec((8, 128), lambda i: (0, 0))],
        out_shape=[jax.ShapeDtypeStruct((n, m), jnp.float32),
                   jax.ShapeDtypeStruct((8, 128), jnp.float32)],
    )(x, wt)


def _recip_body(s_ref, o_ref):
    o_ref[...] = 1.0 / (s_ref[0:1, :] + s_ref[1:2, :] + 1e-16)


def _recip(s_part):
    out = pl.pallas_call(
        _recip_body,
        out_shape=jax.ShapeDtypeStruct((1, _NP), jnp.float32),
    )(s_part)
    return out.reshape(_NP)


def _add2r_body(a_ref, b_ref, r_ref, o_ref):
    o_ref[...] = (a_ref[0] + b_ref[0]) * r_ref[...]


def _add2r(p, r, bm=2048):
    _, n, m = p.shape
    return pl.pallas_call(
        _add2r_body,
        grid=(n // bm,),
        in_specs=[pl.BlockSpec((1, bm, m), lambda i: (0, i, 0)),
                  pl.BlockSpec((1, bm, m), lambda i: (1, i, 0)),
                  pl.BlockSpec((bm, 1), lambda i: (i, 0))],
        out_specs=pl.BlockSpec((bm, m), lambda i: (i, 0)),
        out_shape=jax.ShapeDtypeStruct((n, m), jnp.float32),
    )(p, p, r)


@functools.lru_cache(maxsize=None)
def _gat_phase1(dd):
    mesh = plsc.VectorSubcoreMesh(core_axis_name="c", subcore_axis_name="s")

    def body(xl_hbm, xr_hbm, src_hbm, dst_hbm, attb_hbm, mg_hbm, zs_hbm,
             ex_hbm, s_part_hbm,
             attb_v, mg_v, src2d, dst2d, ex2d,
             xl0, xl1, xr0, xr1, s_sh,
             sem_xl0, sem_xl1, sem_xr0, sem_xr1, sem_sc):
        c = lax.axis_index("c")
        s = lax.axis_index("s")
        wid = s * 2 + c
        pltpu.sync_copy(attb_hbm, attb_v)
        pltpu.sync_copy(mg_hbm, mg_v)
        pltpu.sync_copy(src_hbm.at[pl.ds(wid * _NSUB, _NSUB), :], src2d)
        pltpu.sync_copy(dst_hbm.at[pl.ds(wid * _NSUB, _NSUB), :], dst2d)
        pltpu.sync_copy(zs_hbm, s_sh.at[pl.ds(s * 640, 640)])
        plsc.subcore_barrier()
        mgv = mg_v[...]
        iota = lax.iota(jnp.int32, 16)
        xls, xrs = [xl0, xl1], [xr0, xr1]
        sem_xl, sem_xr = [sem_xl0, sem_xl1], [sem_xr0, sem_xr1]

        descs = {}

        def start(j):
            b = j % 2
            descs[j] = (
                pltpu.async_copy(xl_hbm.at[src2d.at[j]], xls[b], sem_xl[b]),
                pltpu.async_copy(xr_hbm.at[dst2d.at[j]], xrs[b], sem_xr[b]),
            )

        start(0)
        start(1)
        sc_descs = {}
        for j in range(_NSUB):
            d1, d2 = descs.pop(j)
            d1.wait()
            d2.wait()
            xlb, xrb = xls[j % 2], xrs[j % 2]

            def group(g, carry2, j=j, xlb=xlb, xrb=xrb):
                rows = iota + g * 16

                def dstep(d, acc):
                    dvec = (d + iota) & (dd - 1)
                    a = plsc.load_gather(xlb, [rows, dvec])
                    b2 = plsc.load_gather(xrb, [rows, dvec])
                    v = a + b2
                    lr = jnp.maximum(v, 0.2 * v)
                    return acc + lr * attb_v[pl.ds(d, 16)]

                logit = lax.fori_loop(0, dd, dstep,
                                      jnp.zeros((16,), jnp.float32), unroll=8)
                ex2d[j, pl.ds(g * 16, 16)] = jnp.exp(logit - mgv)
                return carry2

            lax.fori_loop(0, _SUB // 16, group, 0)
            if j + 2 < _NSUB:
                start(j + 2)
            sc_descs[j] = pltpu.async_copy(ex2d.at[j], s_sh.at[dst2d.at[j]],
                                           sem_sc, add=True)
            if j >= 8:
                sc_descs.pop(j - 8).wait()
        for dsc in sc_descs.values():
            dsc.wait()
        pltpu.sync_copy(ex2d, ex_hbm.at[pl.ds(wid * _NSUB, _NSUB), :])
        plsc.subcore_barrier()

        @pl.when(s == 0)
        def _():
            pltpu.sync_copy(s_sh, s_part_hbm.at[c])

    return pl.kernel(
        body,
        compiler_params=pltpu.CompilerParams(use_tc_tiling_on_sc=False,
                                             needs_layout_passes=False),
        out_type=[jax.ShapeDtypeStruct((_EP // _SUB, _SUB), jnp.float32),
                  jax.ShapeDtypeStruct((2, _NP), jnp.float32)],
        mesh=mesh,
        scratch_types=[
            pltpu.VMEM((dd + 16,), jnp.float32),
            pltpu.VMEM((16,), jnp.float32),
            pltpu.VMEM((_NSUB, _SUB), jnp.int32),
            pltpu.VMEM((_NSUB, _SUB), jnp.int32),
            pltpu.VMEM((_NSUB, _SUB), jnp.float32),
            pltpu.VMEM((_SUB, dd), jnp.float32),
            pltpu.VMEM((_SUB, dd), jnp.float32),
            pltpu.VMEM((_SUB, dd), jnp.float32),
            pltpu.VMEM((_SUB, dd), jnp.float32),
            pltpu.VMEM_SHARED((_NP,), jnp.float32),
            pltpu.SemaphoreType.DMA,
            pltpu.SemaphoreType.DMA,
            pltpu.SemaphoreType.DMA,
            pltpu.SemaphoreType.DMA,
            pltpu.SemaphoreType.DMA,
        ],
    )


@functools.lru_cache(maxsize=None)
def _gat_phase2(dd):
    mesh = plsc.VectorSubcoreMesh(core_axis_name="c", subcore_axis_name="s")
    nb = 2 if dd == 64 else 1

    def body(xl_hbm, src_hbm, dst_hbm, ex_hbm, zo_hbm, out_part_hbm, *scr):
        if nb == 2:
            (src2d, dst2d, ex2d, xl0, xl1, sc0, sc1, out_sh,
             sem_xl0, sem_xl1, sem_sc0, sem_sc1) = scr
            xls, scs = [xl0, xl1], [sc0, sc1]
            sem_xl, sem_sc = [sem_xl0, sem_xl1], [sem_sc0, sem_sc1]
        else:
            (src2d, dst2d, ex2d, xl0, sc0, out_sh, sem_xl0, sem_sc0) = scr
            xls, scs = [xl0], [sc0]
            sem_xl, sem_sc = [sem_xl0], [sem_sc0]
        c = lax.axis_index("c")
        s = lax.axis_index("s")
        wid = s * 2 + c
        pltpu.sync_copy(src_hbm.at[pl.ds(wid * _NSUB, _NSUB), :], src2d)
        pltpu.sync_copy(dst_hbm.at[pl.ds(wid * _NSUB, _NSUB), :], dst2d)
        pltpu.sync_copy(ex_hbm.at[pl.ds(wid * _NSUB, _NSUB), :], ex2d)
        pltpu.sync_copy(zo_hbm, out_sh.at[pl.ds(s * 640, 640), :])
        plsc.subcore_barrier()
        iota = lax.iota(jnp.int32, 16)

        descs = {}

        def start(j):
            b = j % nb
            descs[j] = pltpu.async_copy(xl_hbm.at[src2d.at[j]], xls[b],
                                        sem_xl[b])

        for t in range(nb):
            start(t)
        sc_descs = {}
        for j in range(_NSUB):
            descs.pop(j).wait()
            b = j % nb
            if (j - nb) in sc_descs:
                sc_descs.pop(j - nb).wait()
            xlb, scb = xls[b], scs[b]

            def group(g, carry2, j=j, xlb=xlb, scb=scb):
                rows = iota + g * 16
                alpha = ex2d[j, pl.ds(g * 16, 16)]

                def dstep(d, carry3):
                    dvec = (d + iota) & (dd - 1)
                    valv = plsc.load_gather(xlb, [rows, dvec])
                    plsc.store_scatter(scb, [rows, dvec], valv * alpha)
                    return carry3

                lax.fori_loop(0, dd, dstep, 0, unroll=8)
                return carry2

            lax.fori_loop(0, _SUB // 16, group, 0)
            sc_descs[j] = pltpu.async_copy(scb, out_sh.at[dst2d.at[j]],
                                           sem_sc[b], add=True)
            if j + nb < _NSUB:
                start(j + nb)
        for dsc in sc_descs.values():
            dsc.wait()
        plsc.subcore_barrier()
        pltpu.sync_copy(out_sh.at[pl.ds(s * 640, 640), :],
                        out_part_hbm.at[c, pl.ds(s * 640, 640), :])

    scratch = [
        pltpu.VMEM((_NSUB, _SUB), jnp.int32),
        pltpu.VMEM((_NSUB, _SUB), jnp.int32),
        pltpu.VMEM((_NSUB, _SUB), jnp.float32),
    ]
    scratch += [pltpu.VMEM((_SUB, dd), jnp.float32)] * nb
    scratch += [pltpu.VMEM((_SUB, dd), jnp.float32)] * nb
    scratch += [pltpu.VMEM_SHARED((_NP, dd), jnp.float32)]
    scratch += [pltpu.SemaphoreType.DMA] * (2 * nb)
    return pl.kernel(
        body,
        compiler_params=pltpu.CompilerParams(use_tc_tiling_on_sc=False,
                                             needs_layout_passes=False),
        out_type=jax.ShapeDtypeStruct((2, _NP, dd), jnp.float32),
        mesh=mesh,
        scratch_types=scratch,
    )


def _finish_body(zz_ref, vs_ref, rs_ref, w_ref, bb_ref, r1_ref, r2_ref):
    zz = zz_ref[...]
    emb = jnp.maximum(zz[:, :64], 0.0)
    emb_a = jnp.maximum(zz[:, 64:], 0.0)
    vs = vs_ref[...]
    row_sum = rs_ref[:, :1]

    def fin(geblk):
        ge = geblk / row_sum
        nrm = jnp.sqrt(jnp.sum(ge * ge, axis=1, keepdims=True))
        return jax.nn.sigmoid(ge / jnp.maximum(nrm, 1e-12))

    g = fin(vs[:, :64])
    g_a = fin(vs[:, 64:])
    w = w_ref[...]
    a1 = jnp.dot(emb, w, preferred_element_type=jnp.float32)
    a2 = jnp.dot(emb_a, w, preferred_element_type=jnp.float32)
    b = bb_ref[0, 0]
    sc_1 = jnp.sum(a1 * g, axis=1, keepdims=True) + b
    sc_2 = jnp.sum(a2 * g, axis=1, keepdims=True) + b
    sc_1a = jnp.sum(a2 * g_a, axis=1, keepdims=True) + b
    sc_2a = jnp.sum(a1 * g_a, axis=1, keepdims=True) + b
    r1_ref[...] = jax.nn.sigmoid(jnp.concatenate([sc_1, sc_2], axis=1))
    r2_ref[...] = jax.nn.sigmoid(jnp.concatenate([sc_1a, sc_2a], axis=1))


def _finish(zz, vs, rs, w_disc, b_disc, bm=400):
    n = zz.shape[0]
    bb = jnp.broadcast_to(b_disc.reshape(1, 1), (8, 128))
    return pl.pallas_call(
        _finish_body,
        grid=(n // bm,),
        in_specs=[pl.BlockSpec((bm, 128), lambda i: (i, 0)),
                  pl.BlockSpec((bm, 128), lambda i: (i, 0)),
                  pl.BlockSpec((bm, 8), lambda i: (i, 0)),
                  pl.BlockSpec((64, 64), lambda i: (0, 0)),
                  pl.BlockSpec((8, 128), lambda i: (0, 0))],
        out_specs=[pl.BlockSpec((bm, 2), lambda i: (i, 0)),
                   pl.BlockSpec((bm, 2), lambda i: (i, 0))],
        out_shape=[jax.ShapeDtypeStruct((n, 2), jnp.float32),
                   jax.ShapeDtypeStruct((n, 2), jnp.float32)],
    )(zz, vs, rs, w_disc, bb)


def _gatv2_sc(xl, xr, mg, att, src2d, dst2d):
    dd = xl.shape[1]
    attb = jnp.concatenate([att, att[:16]])
    mg16 = jnp.full((16,), mg, jnp.float32)
    zs = jnp.zeros((640,), jnp.float32)
    zo = jnp.zeros((640, 64), jnp.float32)
    ex, s_part = _gat_phase1(dd)(xl, xr, src2d, dst2d, attb, mg16, zs)
    r = _recip(s_part).reshape(_NP, 1)
    outs = []
    for lo in range(0, dd, 64):
        part = _gat_phase2(64)(xl[:, lo:lo + 64] + 0.0,
                               src2d, dst2d, ex, zo)
        outs.append(_add2r(part, r))
    if len(outs) == 1:
        return outs[0]
    return jnp.concatenate(outs, axis=1)


def kernel(feat, feat_a, graph_neigh, edge_index, adj,
           Wl_zip, Wr_zip, att_zip, Wl_eco, Wr_eco, att_eco, W_disc, b_disc):
    n = feat.shape[0]
    e = edge_index.shape[1]
    srcp = jnp.concatenate(
        [edge_index[0], jnp.full((_EP - e,), _PAD_IDX, jnp.int32)]
    ).reshape(_EP // _SUB, _SUB)
    dstp = jnp.concatenate(
        [edge_index[1], jnp.full((_EP - e,), _PAD_IDX, jnp.int32)]
    ).reshape(_EP // _SUB, _SUB)

    featp = jnp.pad(feat, ((0, _NP - n), (0, 0)))
    featap = jnp.pad(feat_a, ((0, _NP - n), (0, 0)))

    att_nz = jnp.sqrt(jnp.sum(att_zip * att_zip))
    xl1, nl1 = _proj(featp, Wl_zip.T)
    xr1, nr1 = _proj(featp, Wr_zip.T)
    xl2, nl2 = _proj(featap, Wl_zip.T)
    xr2, nr2 = _proj(featap, Wr_zip.T)
    mg1 = att_nz * (jnp.sqrt(jnp.max(nl1)) + jnp.sqrt(jnp.max(nr1)))
    mg2 = att_nz * (jnp.sqrt(jnp.max(nl2)) + jnp.sqrt(jnp.max(nr2)))

    z = _gatv2_sc(xl1, xr1, mg1, att_zip, srcp, dstp)
    z_a = _gatv2_sc(xl2, xr2, mg2, att_zip, srcp, dstp)

    zz = _mm(adj, jnp.concatenate([z[:n], z_a[:n]], axis=1))
    z2 = zz[:, :64]
    z_a2 = zz[:, 64:]

    z2p = jnp.pad(z2, ((0, _NP - n), (0, 0)))
    att_ne = jnp.sqrt(jnp.sum(att_eco * att_eco))
    xl3, nl3 = _proj(z2p, Wl_eco.T)
    xr3, nr3 = _proj(z2p, Wr_eco.T)
    mg3 = att_ne * (jnp.sqrt(jnp.max(nl3)) + jnp.sqrt(jnp.max(nr3)))
    h = _gatv2_sc(xl3, xr3, mg3, att_eco, srcp, dstp)
    h2 = _mm(adj, h[:n])

    vs, rs = _mm_rowsum(graph_neigh, jax.nn.relu(zz))
    ret, ret_a = _finish(zz, vs, rs, W_disc, b_disc)

    return (z2, h2, ret, ret_a)
